# baseline (speedup 1.0000x reference)
import jax
import jax.numpy as jnp
from jax.experimental import pallas as pl
from jax.experimental.pallas import tpu as pltpu

_B, _C, _H, _W = 32, 3, 512, 512
_NROWS = 16
_LANES = _H * _W // _NROWS
_NUM_PIXEL = float(_H * _W + 256)


_BB = 1


def _bands(a):
    return jnp.concatenate(
        [a[16 * g:16 * (g + 1), :] for g in range(_H // 16)], axis=1)


def _hist_kernel(x_ref, out_ref):
    one = jnp.bfloat16(1.0)
    zero = jnp.bfloat16(0.0)
    for b in range(_BB):
        t = ((x_ref[b, 0] + x_ref[b, 1]) + x_ref[b, 2]) * 85.0
        tf = jnp.floor(t)
        hh = jnp.floor(t * 0.0625)
        ll = tf - hh * 16.0
        hhb = _bands(hh.astype(jnp.bfloat16))
        llb = _bands(ll.astype(jnp.bfloat16))
        lhs = jnp.concatenate(
            [jnp.where(hhb == jnp.bfloat16(h), one, zero) for h in range(16)],
            axis=0).astype(jnp.float8_e4m3fn)
        rhs = jnp.concatenate(
            [jnp.where(llb == jnp.bfloat16(l), one, zero) for l in range(16)],
            axis=0).astype(jnp.float8_e4m3fn)
        out_ref[b] = jax.lax.dot_general(
            lhs, rhs, (((1,), (1,)), ((), ())),
            preferred_element_type=jnp.float32)


_LBATCH = 8
_LSTEPS = _B // _LBATCH


def _loss_kernel(p_ref, o_ref, acc_ref):
    i = pl.program_id(0)

    @pl.when(i == 0)
    def _init():
        acc_ref[0] = 0.0

    si = jax.lax.broadcasted_iota(jnp.int32, (256, 256), 0)
    li = jax.lax.broadcasted_iota(jnp.int32, (256, 256), 1)
    mask = (si & 15) == (li & 15)
    gi = jax.lax.broadcasted_iota(jnp.int32, (256, 16), 0)
    gh = jax.lax.broadcasted_iota(jnp.int32, (256, 16), 1)
    g = ((gi >> 4) == gh).astype(jnp.float32)
    bi = jax.lax.broadcasted_iota(jnp.int32, (16, 16), 0)
    bj = jax.lax.broadcasted_iota(jnp.int32, (16, 16), 1)
    quirk = (bi == 0) & (bj == 0)

    tot = jnp.float32(0.0)
    for b in range(_LBATCH):
        pm = jnp.where(mask, p_ref[b], 0.0)
        r = jax.lax.dot_general(
            g, pm, (((0,), (0,)), ((), ())),
            preferred_element_type=jnp.float32)
        counts = jax.lax.dot_general(
            r, g, (((1,), (0,)), ((), ())),
            preferred_element_type=jnp.float32)
        c = jnp.where(quirk, float(_H * _W), counts)
        prob = (c + 1.0) * (1.0 / _NUM_PIXEL)
        ent = prob * jnp.log(prob)
        tot = tot + jnp.sum(ent)
    acc_ref[0] += tot * (1.0 / _B)

    @pl.when(i == _LSTEPS - 1)
    def _finish():
        o_ref[0, 0] = acc_ref[0]


@jax.jit
def kernel(x):
    pmat = pl.pallas_call(
        _hist_kernel,
        out_shape=jax.ShapeDtypeStruct((_B, 256, 256), jnp.float32),
        grid=(_B // _BB,),
        in_specs=[pl.BlockSpec((_BB, _C, _H, _W), lambda b: (b, 0, 0, 0))],
        out_specs=pl.BlockSpec((_BB, 256, 256), lambda b: (b, 0, 0)),
        compiler_params=pltpu.CompilerParams(
            dimension_semantics=("parallel",)),
        name="hist256_mxu",
    )(x)
    loss = pl.pallas_call(
        _loss_kernel,
        out_shape=jax.ShapeDtypeStruct((1, 1), jnp.float32),
        grid=(_LSTEPS,),
        in_specs=[pl.BlockSpec((_LBATCH, 256, 256), lambda i: (i, 0, 0))],
        out_specs=pl.BlockSpec(memory_space=pltpu.SMEM),
        scratch_shapes=[pltpu.SMEM((1,), jnp.float32)],
        compiler_params=pltpu.CompilerParams(
            dimension_semantics=("arbitrary",)),
        name="hist_entropy",
    )(pmat)
    return loss[0, 0]


# speedup vs baseline: 1.0659x; 1.0659x over previous
import jax
import jax.numpy as jnp
from jax.experimental import pallas as pl
from jax.experimental.pallas import tpu as pltpu

_B, _C, _H, _W = 32, 3, 512, 512
_NROWS = 16
_LANES = _H * _W // _NROWS
_NUM_PIXEL = float(_H * _W + 256)


_BB = 2


def _bands(a):
    return jnp.concatenate(
        [a[16 * g:16 * (g + 1), :] for g in range(_H // 16)], axis=1)


def _hist_kernel(x_ref, out_ref):
    one = jnp.bfloat16(1.0)
    zero = jnp.bfloat16(0.0)
    for b in range(_BB):
        t = ((x_ref[b, 0] + x_ref[b, 1]) + x_ref[b, 2]) * 85.0
        tf = jnp.floor(t)
        hh = jnp.floor(t * 0.0625)
        ll = tf - hh * 16.0
        hhb = _bands(hh.astype(jnp.bfloat16))
        llb = _bands(ll.astype(jnp.bfloat16))
        lhs = jnp.concatenate(
            [jnp.where(hhb == jnp.bfloat16(h), one, zero) for h in range(16)],
            axis=0).astype(jnp.float8_e4m3fn)
        rhs = jnp.concatenate(
            [jnp.where(llb == jnp.bfloat16(l), one, zero) for l in range(16)],
            axis=0).astype(jnp.float8_e4m3fn)
        out_ref[b] = jax.lax.dot_general(
            lhs, rhs, (((1,), (1,)), ((), ())),
            preferred_element_type=jnp.float32)


_LBATCH = 8
_LSTEPS = _B // _LBATCH


def _loss_kernel(p_ref, o_ref, acc_ref):
    i = pl.program_id(0)

    @pl.when(i == 0)
    def _init():
        acc_ref[0] = 0.0

    si = jax.lax.broadcasted_iota(jnp.int32, (256, 256), 0)
    li = jax.lax.broadcasted_iota(jnp.int32, (256, 256), 1)
    mask = (si & 15) == (li & 15)
    gi = jax.lax.broadcasted_iota(jnp.int32, (256, 16), 0)
    gh = jax.lax.broadcasted_iota(jnp.int32, (256, 16), 1)
    g = ((gi >> 4) == gh).astype(jnp.float32)
    bi = jax.lax.broadcasted_iota(jnp.int32, (16, 16), 0)
    bj = jax.lax.broadcasted_iota(jnp.int32, (16, 16), 1)
    quirk = (bi == 0) & (bj == 0)

    tot = jnp.float32(0.0)
    for b in range(_LBATCH):
        pm = jnp.where(mask, p_ref[b], 0.0)
        r = jax.lax.dot_general(
            g, pm, (((0,), (0,)), ((), ())),
            preferred_element_type=jnp.float32)
        counts = jax.lax.dot_general(
            r, g, (((1,), (0,)), ((), ())),
            preferred_element_type=jnp.float32)
        c = jnp.where(quirk, float(_H * _W), counts)
        prob = (c + 1.0) * (1.0 / _NUM_PIXEL)
        ent = prob * jnp.log(prob)
        tot = tot + jnp.sum(ent)
    acc_ref[0] += tot * (1.0 / _B)

    @pl.when(i == _LSTEPS - 1)
    def _finish():
        o_ref[0, 0] = acc_ref[0]


@jax.jit
def kernel(x):
    pmat = pl.pallas_call(
        _hist_kernel,
        out_shape=jax.ShapeDtypeStruct((_B, 256, 256), jnp.float32),
        grid=(_B // _BB,),
        in_specs=[pl.BlockSpec((_BB, _C, _H, _W), lambda b: (b, 0, 0, 0))],
        out_specs=pl.BlockSpec((_BB, 256, 256), lambda b: (b, 0, 0)),
        compiler_params=pltpu.CompilerParams(
            dimension_semantics=("parallel",)),
        name="hist256_mxu",
    )(x)
    loss = pl.pallas_call(
        _loss_kernel,
        out_shape=jax.ShapeDtypeStruct((1, 1), jnp.float32),
        grid=(_LSTEPS,),
        in_specs=[pl.BlockSpec((_LBATCH, 256, 256), lambda i: (i, 0, 0))],
        out_specs=pl.BlockSpec(memory_space=pltpu.SMEM),
        scratch_shapes=[pltpu.SMEM((1,), jnp.float32)],
        compiler_params=pltpu.CompilerParams(
            dimension_semantics=("arbitrary",)),
        name="hist_entropy",
    )(pmat)
    return loss[0, 0]


# speedup vs baseline: 1.0995x; 1.0315x over previous
import jax
import jax.numpy as jnp
from jax.experimental import pallas as pl
from jax.experimental.pallas import tpu as pltpu

_B, _C, _H, _W = 32, 3, 512, 512
_NROWS = 16
_LANES = _H * _W // _NROWS
_NUM_PIXEL = float(_H * _W + 256)


_BB = 2


def _bands(a):
    return jnp.concatenate(
        [a[16 * g:16 * (g + 1), :] for g in range(_H // 16)], axis=1)


def _hist_kernel(x_ref, out_ref):
    one = jnp.bfloat16(1.0)
    zero = jnp.bfloat16(0.0)
    for b in range(_BB):
        t = ((x_ref[b, 0] + x_ref[b, 1]) + x_ref[b, 2]) * 85.0
        tf = jnp.floor(t)
        hh = jnp.floor(t * 0.0625)
        ll = tf - hh * 16.0
        hhb = _bands(hh.astype(jnp.bfloat16))
        llb = _bands(ll.astype(jnp.bfloat16))
        lhs = jnp.concatenate(
            [jnp.where(hhb == jnp.bfloat16(h), one, zero) for h in range(16)],
            axis=0).astype(jnp.float8_e4m3fn)
        rhs = jnp.concatenate(
            [jnp.where(llb == jnp.bfloat16(l), one, zero) for l in range(16)],
            axis=0).astype(jnp.float8_e4m3fn)
        out_ref[b] = jax.lax.dot_general(
            lhs, rhs, (((1,), (1,)), ((), ())),
            preferred_element_type=jnp.float32)


_LBATCH = 8
_LSTEPS = _B // _LBATCH


def _loss_kernel(p_ref, o_ref, acc_ref):
    i = pl.program_id(0)

    @pl.when(i == 0)
    def _init():
        acc_ref[0] = 0.0

    si = jax.lax.broadcasted_iota(jnp.int32, (256, 256), 0)
    li = jax.lax.broadcasted_iota(jnp.int32, (256, 256), 1)
    mask = (si & 15) == (li & 15)
    gi = jax.lax.broadcasted_iota(jnp.int32, (256, 16), 0)
    gh = jax.lax.broadcasted_iota(jnp.int32, (256, 16), 1)
    g = ((gi >> 4) == gh).astype(jnp.float32)
    bi = jax.lax.broadcasted_iota(jnp.int32, (16, 16), 0)
    bj = jax.lax.broadcasted_iota(jnp.int32, (16, 16), 1)
    quirk = (bi == 0) & (bj == 0)

    tot = jnp.float32(0.0)
    for b in range(_LBATCH):
        pm = jnp.where(mask, p_ref[b], 0.0)
        r = pm.reshape(16, 16, 256).sum(axis=1)
        counts = jax.lax.dot_general(
            r, g, (((1,), (0,)), ((), ())),
            preferred_element_type=jnp.float32)
        c = jnp.where(quirk, float(_H * _W), counts)
        prob = (c + 1.0) * (1.0 / _NUM_PIXEL)
        ent = prob * jnp.log(prob)
        tot = tot + jnp.sum(ent)
    acc_ref[0] += tot * (1.0 / _B)

    @pl.when(i == _LSTEPS - 1)
    def _finish():
        o_ref[0, 0] = acc_ref[0]


@jax.jit
def kernel(x):
    pmat = pl.pallas_call(
        _hist_kernel,
        out_shape=jax.ShapeDtypeStruct((_B, 256, 256), jnp.float32),
        grid=(_B // _BB,),
        in_specs=[pl.BlockSpec((_BB, _C, _H, _W), lambda b: (b, 0, 0, 0))],
        out_specs=pl.BlockSpec((_BB, 256, 256), lambda b: (b, 0, 0)),
        compiler_params=pltpu.CompilerParams(
            dimension_semantics=("parallel",)),
        name="hist256_mxu",
    )(x)
    loss = pl.pallas_call(
        _loss_kernel,
        out_shape=jax.ShapeDtypeStruct((1, 1), jnp.float32),
        grid=(_LSTEPS,),
        in_specs=[pl.BlockSpec((_LBATCH, 256, 256), lambda i: (i, 0, 0))],
        out_specs=pl.BlockSpec(memory_space=pltpu.SMEM),
        scratch_shapes=[pltpu.SMEM((1,), jnp.float32)],
        compiler_params=pltpu.CompilerParams(
            dimension_semantics=("arbitrary",)),
        name="hist_entropy",
    )(pmat)
    return loss[0, 0]
